# speedup vs baseline: 3.5796x; 3.5796x over previous
import functools

import jax
import jax.numpy as jnp
from jax import lax
from jax.experimental import pallas as pl
from jax.experimental.pallas import tpu as pltpu
from jax.experimental.pallas import tpu_sc as plsc

D = 768
TYPE_VOCAB = 2

_NC, _NS = 2, 16
_NW = _NC * _NS
_CHUNK = 128


def _sc_gather(table, flat_ids):
    n = flat_ids.shape[0]
    b_per_w = n // _NW
    n_chunks = b_per_w // _CHUNK
    mesh = plsc.VectorSubcoreMesh(core_axis_name="c", subcore_axis_name="s")

    @functools.partial(
        pl.kernel,
        mesh=mesh,
        out_type=jax.ShapeDtypeStruct((n, D), jnp.float32),
        scratch_types=[
            pltpu.VMEM((_CHUNK,), jnp.int32),
            pltpu.VMEM((_CHUNK, D), jnp.float32),
            pltpu.SemaphoreType.DMA,
        ],
    )
    def k(table_hbm, idx_hbm, out_hbm, idx_v, rows_v, sem):
        wid = lax.axis_index("s") * _NC + lax.axis_index("c")
        base = wid * b_per_w

        @pl.loop(0, n_chunks)
        def _(c):
            off = base + c * _CHUNK
            pltpu.sync_copy(idx_hbm.at[pl.ds(off, _CHUNK)], idx_v)
            pltpu.async_copy(table_hbm.at[idx_v], rows_v, sem).wait()
            pltpu.sync_copy(rows_v, out_hbm.at[pl.ds(off, _CHUNK)])

    return k(table, flat_ids)


_BB = 8


def _tc_body(g_ref, tt_ref, pos_ref, comb_ref, w_ref, b_ref, o_ref):
    x = g_ref[...]
    tt = tt_ref[...]
    oh = (tt[..., None] == lax.broadcasted_iota(
        jnp.int32, tt.shape + (16,), 2)).astype(jnp.float32)
    te = lax.dot_general(
        oh.reshape(-1, 16), comb_ref[...],
        (((1,), (0,)), ((), ())),
        precision=lax.Precision.HIGHEST,
        preferred_element_type=jnp.float32,
    )
    x = x + pos_ref[...][None] + te.reshape(x.shape)
    u = jnp.mean(x, axis=-1, keepdims=True)
    s = jnp.mean((x - u) ** 2, axis=-1, keepdims=True)
    y = (x - u) / jnp.sqrt(s + 1e-12)
    o_ref[...] = w_ref[...] * y + b_ref[...]


def kernel(input_ids, token_type_ids, word_emb, pos_emb, type_emb,
           type_ext_emb, ln_w, ln_b):
    B, S = input_ids.shape
    ids = input_ids.astype(jnp.int32).reshape(-1)
    gathered = _sc_gather(word_emb, ids).reshape(B, S, D)

    tt = token_type_ids.astype(jnp.int32)
    pos_s = pos_emb[:S]
    comb = jnp.concatenate(
        [type_emb, type_ext_emb,
         jnp.zeros((16 - TYPE_VOCAB - type_ext_emb.shape[0], D), jnp.float32)],
        axis=0)

    return pl.pallas_call(
        _tc_body,
        grid=(B // _BB,),
        in_specs=[
            pl.BlockSpec((_BB, S, D), lambda i: (i, 0, 0)),
            pl.BlockSpec((_BB, S), lambda i: (i, 0)),
            pl.BlockSpec((S, D), lambda i: (0, 0)),
            pl.BlockSpec((16, D), lambda i: (0, 0)),
            pl.BlockSpec((1, D), lambda i: (0, 0)),
            pl.BlockSpec((1, D), lambda i: (0, 0)),
        ],
        out_specs=pl.BlockSpec((_BB, S, D), lambda i: (i, 0, 0)),
        out_shape=jax.ShapeDtypeStruct((B, S, D), jnp.float32),
    )(gathered, tt, pos_s, comb, ln_w.reshape(1, D), ln_b.reshape(1, D))
